# baseline (speedup 1.0000x reference)
import functools

import jax
import jax.numpy as jnp
from jax import lax
from jax.experimental import pallas as pl
from jax.experimental.pallas import tpu as pltpu
from jax.experimental.pallas import tpu_sc as plsc

B = 16384
E = 32
NC = 2
NS = 16
NW = NC * NS
RPW = B // NW
CHUNK = 128
NCHUNK = RPW // CHUNK
NT = 7


def _fm_body(idx_hbm, users, items, cats, prices, out_p, out_n,
             idx_v, rows_v, score_v, sem):
    wid = lax.axis_index("s") * NC + lax.axis_index("c")
    base = wid * RPW
    tables = (users, items, items, cats, cats, prices, prices)

    pltpu.sync_copy(idx_hbm.at[wid], idx_v)
    iota16 = lax.iota(jnp.int32, 16)

    def chunk_body(c, carry):
        copies = [
            pltpu.async_copy(tables[t].at[idx_v.at[t, c]], rows_v.at[t], sem)
            for t in range(NT)
        ]
        for cp in copies:
            cp.wait()

        def group_body(g, carry2):
            row16 = g * 16 + iota16
            accP = jnp.zeros((16,), jnp.float32)
            sqP = jnp.zeros((16,), jnp.float32)
            accN = jnp.zeros((16,), jnp.float32)
            sqN = jnp.zeros((16,), jnp.float32)
            accU = jnp.zeros((16,), jnp.float32)
            for e in range(E):
                cole = jnp.full((16,), e, jnp.int32)
                u = plsc.load_gather(rows_v.at[0], [row16, cole])
                ip = plsc.load_gather(rows_v.at[1], [row16, cole])
                inn = plsc.load_gather(rows_v.at[2], [row16, cole])
                cpv = plsc.load_gather(rows_v.at[3], [row16, cole])
                cnv = plsc.load_gather(rows_v.at[4], [row16, cole])
                ppv = plsc.load_gather(rows_v.at[5], [row16, cole])
                pnv = plsc.load_gather(rows_v.at[6], [row16, cole])
                accU = accU + u * u
                sp = (u + ip) + (cpv + ppv)
                accP = accP + sp * sp
                sqP = sqP + (ip * ip + (cpv * cpv + ppv * ppv))
                sn = (u + inn) + (cnv + pnv)
                accN = accN + sn * sn
                sqN = sqN + (inn * inn + (cnv * cnv + pnv * pnv))
            off = c * CHUNK + g * 16
            score_v[0, pl.ds(off, 16)] = 0.5 * (accP - sqP - accU)
            score_v[1, pl.ds(off, 16)] = 0.5 * (accN - sqN - accU)
            return carry2

        lax.fori_loop(0, CHUNK // 16, group_body, 0)
        return carry

    lax.fori_loop(0, NCHUNK, chunk_body, 0)
    pltpu.sync_copy(score_v.at[0], out_p.at[pl.ds(base, RPW)])
    pltpu.sync_copy(score_v.at[1], out_n.at[pl.ds(base, RPW)])


@functools.partial(
    pl.kernel,
    out_type=[
        jax.ShapeDtypeStruct((B,), jnp.float32),
        jax.ShapeDtypeStruct((B,), jnp.float32),
    ],
    mesh=plsc.VectorSubcoreMesh(core_axis_name="c", subcore_axis_name="s"),
    compiler_params=pltpu.CompilerParams(
        needs_layout_passes=False, use_tc_tiling_on_sc=False
    ),
    scratch_types=[
        pltpu.VMEM((NT, NCHUNK, CHUNK), jnp.int32),
        pltpu.VMEM((NT, CHUNK, E), jnp.float32),
        pltpu.VMEM((2, RPW), jnp.float32),
        pltpu.SemaphoreType.DMA,
    ],
)
def _fm_kernel(idx_hbm, users, items, cats, prices, out_p, out_n,
               idx_v, rows_v, score_v, sem):
    _fm_body(idx_hbm, users, items, cats, prices, out_p, out_n,
             idx_v, rows_v, score_v, sem)


def kernel(user, item_p, item_n, cat_p, cat_n, price_p, price_n,
           users, items, cats, prices):
    idx = jnp.stack(
        [user, item_p, item_n, cat_p, cat_n, price_p, price_n]
    ).astype(jnp.int32)
    idx = idx.reshape(NT, NW, NCHUNK, CHUNK).transpose(1, 0, 2, 3)
    p_score, n_score = _fm_kernel(idx, users, items, cats, prices)
    return (p_score, n_score)
